# Initial kernel scaffold; baseline (speedup 1.0000x reference)
import jax
import jax.numpy as jnp
from jax.experimental import pallas as pl


def kernel(query_image, support_image, support_target, W, n_way):
    raise NotImplementedError("write your pallas kernel here")


# speedup vs baseline: 1.8438x; 1.8438x over previous
import functools

import jax
import jax.numpy as jnp
from jax.experimental import pallas as pl
from jax.experimental.pallas import tpu as pltpu

Q, NS, D_IN, D_EMB, NWAY = 16384, 6400, 2048, 512, 64
SB = 128
QB = 512


def _proto_kernel(s_ref, t_ref, w_ref, proto_ref, acc_ref):
    i = pl.program_id(0)

    @pl.when(i == 0)
    def _init():
        acc_ref[...] = jnp.zeros_like(acc_ref)

    emb = jnp.dot(s_ref[...], w_ref[...], preferred_element_type=jnp.float32)
    t = t_ref[0, 0, :]
    oh = (t[:, None] == jax.lax.broadcasted_iota(jnp.int32, (SB, NWAY), 1))
    oh = oh.astype(jnp.float32)
    acc_ref[...] += jax.lax.dot_general(
        oh, emb, (((0,), (0,)), ((), ())),
        preferred_element_type=jnp.float32,
        precision=jax.lax.Precision.HIGHEST)

    @pl.when(i == pl.num_programs(0) - 1)
    def _finalize():
        sums = acc_ref[...]
        norm = jnp.sqrt(jnp.sum(sums * sums, axis=1, keepdims=True))
        proto_ref[...] = sums / jnp.maximum(norm, 1e-12)


def _classify_kernel(q_ref, w_ref, proto_ref, out_ref):
    qe = jnp.dot(q_ref[...], w_ref[...], preferred_element_type=jnp.float32)
    proto = proto_ref[...]
    qp = jax.lax.dot_general(
        qe, proto, (((1,), (1,)), ((), ())),
        preferred_element_type=jnp.float32)
    q2 = jnp.sum(qe * qe, axis=1, keepdims=True)
    m2 = jnp.sum(proto * proto, axis=1)[None, :]
    d2 = jnp.maximum(q2 + m2 - 2.0 * qp, 1e-12)
    dist = jnp.sqrt(d2)
    dist = dist * dist
    out_ref[...] = jnp.argmin(dist, axis=1).astype(jnp.int32)


def kernel(query_image, support_image, support_target, W, n_way):
    t3 = support_target.astype(jnp.int32).reshape(NS // SB, 1, SB)

    proto = pl.pallas_call(
        _proto_kernel,
        grid=(NS // SB,),
        in_specs=[
            pl.BlockSpec((SB, D_IN), lambda i: (i, 0)),
            pl.BlockSpec((1, 1, SB), lambda i: (i, 0, 0)),
            pl.BlockSpec((D_IN, D_EMB), lambda i: (0, 0)),
        ],
        out_specs=pl.BlockSpec((NWAY, D_EMB), lambda i: (0, 0)),
        out_shape=jax.ShapeDtypeStruct((NWAY, D_EMB), jnp.float32),
        scratch_shapes=[pltpu.VMEM((NWAY, D_EMB), jnp.float32)],
        compiler_params=pltpu.CompilerParams(
            dimension_semantics=("arbitrary",)),
    )(support_image, t3, W)

    out = pl.pallas_call(
        _classify_kernel,
        grid=(Q // QB,),
        in_specs=[
            pl.BlockSpec((QB, D_IN), lambda i: (i, 0)),
            pl.BlockSpec((D_IN, D_EMB), lambda i: (0, 0)),
            pl.BlockSpec((NWAY, D_EMB), lambda i: (0, 0)),
        ],
        out_specs=pl.BlockSpec((QB,), lambda i: (i,)),
        out_shape=jax.ShapeDtypeStruct((Q,), jnp.int32),
        compiler_params=pltpu.CompilerParams(
            dimension_semantics=("parallel",)),
    )(query_image, W, proto)

    return out
